# baseline (speedup 1.0000x reference)
import functools

import jax
import jax.numpy as jnp
from jax.experimental import pallas as pl
from jax.experimental.pallas import tpu as pltpu
from jax.experimental.pallas import tpu_sc as plsc

D, H, E, K = 768, 2048, 8, 2
T = 2048
BM = 128
G = T * K // BM + E
R = G * BM
CH = 256
SCW = 128


def _router_body(x_ref, rw_ref, rb_ref, w_ref, dest_ref, be_ref, p_ref):
    lg = jnp.dot(x_ref[...], rw_ref[...], preferred_element_type=jnp.float32)
    g = lg.T + rb_ref[...]
    m = jnp.max(g, axis=0, keepdims=True)
    ex = jnp.exp(g - m)
    g = ex / jnp.sum(ex, axis=0, keepdims=True)

    row = jax.lax.broadcasted_iota(jnp.int32, (E, T), 0)
    m1 = jnp.max(g, axis=0, keepdims=True)
    i1 = jnp.min(jnp.where(g == m1, row, E), axis=0, keepdims=True)
    oh1 = row == i1
    gm = jnp.where(oh1, -jnp.inf, g)
    m2 = jnp.max(gm, axis=0, keepdims=True)
    i2 = jnp.min(jnp.where(gm == m2, row, E), axis=0, keepdims=True)
    oh2 = row == i2
    maskf = oh1.astype(jnp.float32) + oh2.astype(jnp.float32)

    r_io = jax.lax.broadcasted_iota(jnp.int32, (CH, CH), 0)
    c_io = jax.lax.broadcasted_iota(jnp.int32, (CH, CH), 1)
    tri = (r_io < c_io).astype(jnp.float32)
    tot = jnp.zeros((E, 1), jnp.float32)
    for i in range(T // CH):
        mblk = maskf[:, i * CH:(i + 1) * CH]
        p_ref[:, i * CH:(i + 1) * CH] = (
            jnp.dot(mblk, tri, preferred_element_type=jnp.float32) + tot)
        tot = tot + jnp.sum(mblk, axis=1, keepdims=True)

    counts = tot
    nbf = jnp.floor((counts + float(BM - 1)) * (1.0 / BM))
    r8 = jax.lax.broadcasted_iota(jnp.int32, (E, E), 0)
    c8 = jax.lax.broadcasted_iota(jnp.int32, (E, E), 1)
    l8 = (c8 <= r8).astype(jnp.float32)
    cum = jnp.dot(l8, nbf, preferred_element_type=jnp.float32)
    bsr = (cum - nbf) * float(BM)

    destf = bsr + p_ref[...]
    dest_ref[0:1, :] = jnp.sum(
        jnp.where(oh1, destf, 0.0), axis=0, keepdims=True).astype(jnp.int32)
    dest_ref[1:2, :] = jnp.sum(
        jnp.where(oh2, destf, 0.0), axis=0, keepdims=True).astype(jnp.int32)
    w_ref[0:1, :] = m1
    w_ref[1:2, :] = m2

    g_io = jax.lax.broadcasted_iota(jnp.int32, (1, 128), 1)
    cum_i = cum.astype(jnp.int32)
    be_acc = jnp.zeros((1, 128), jnp.int32)
    for e in range(E):
        be_acc = be_acc + (g_io >= cum_i[e:e + 1, 0:1]).astype(jnp.int32)
    be_ref[...] = jnp.minimum(be_acc, E - 1)


def _router_call(xf, rw, rb):
    return pl.pallas_call(
        _router_body,
        out_shape=[
            jax.ShapeDtypeStruct((K, T), jnp.float32),
            jax.ShapeDtypeStruct((K, T), jnp.int32),
            jax.ShapeDtypeStruct((1, 128), jnp.int32),
        ],
        scratch_shapes=[pltpu.VMEM((E, T), jnp.float32)],
    )(xf, rw, rb)


_NW = 32
_CHUNK = T // _NW


@functools.lru_cache(maxsize=1)
def _sc_kernels():
    mesh = plsc.VectorSubcoreMesh(core_axis_name="c", subcore_axis_name="s")

    @functools.partial(
        pl.kernel,
        out_type=jax.ShapeDtypeStruct((R, D), jnp.float32),
        mesh=mesh,
        scratch_types=[
            pltpu.VMEM((_CHUNK, D), jnp.float32),
            pltpu.VMEM((_CHUNK,), jnp.int32),
            pltpu.VMEM((_CHUNK,), jnp.int32),
            pltpu.SemaphoreType.DMA,
            pltpu.SemaphoreType.DMA,
        ])
    def dispatch(x_hbm, destT_hbm, xg_hbm, buf, idx0, idx1, sem0, sem1):
        wid = jax.lax.axis_index("s") * 2 + jax.lax.axis_index("c")
        base = wid * _CHUNK
        pltpu.sync_copy(destT_hbm.at[0, pl.ds(base, _CHUNK)], idx0)
        pltpu.sync_copy(destT_hbm.at[1, pl.ds(base, _CHUNK)], idx1)
        pltpu.sync_copy(x_hbm.at[pl.ds(base, _CHUNK)], buf)
        c0 = pltpu.async_copy(buf, xg_hbm.at[idx0], sem0)
        c1 = pltpu.async_copy(buf, xg_hbm.at[idx1], sem1)
        c0.wait()
        c1.wait()

    @functools.partial(
        pl.kernel,
        out_type=jax.ShapeDtypeStruct((K * T, D), jnp.float32),
        mesh=mesh,
        scratch_types=[
            pltpu.VMEM((_CHUNK, D), jnp.float32),
            pltpu.VMEM((_CHUNK, D), jnp.float32),
            pltpu.VMEM((_CHUNK,), jnp.int32),
            pltpu.VMEM((_CHUNK,), jnp.int32),
            pltpu.SemaphoreType.DMA,
            pltpu.SemaphoreType.DMA,
        ])
    def collect(y_hbm, destT_hbm, yk_hbm, buf0, buf1, idx0, idx1, sem0, sem1):
        wid = jax.lax.axis_index("s") * 2 + jax.lax.axis_index("c")
        base = wid * _CHUNK
        pltpu.sync_copy(destT_hbm.at[0, pl.ds(base, _CHUNK)], idx0)
        pltpu.sync_copy(destT_hbm.at[1, pl.ds(base, _CHUNK)], idx1)
        c0 = pltpu.async_copy(y_hbm.at[idx0], buf0, sem0)
        c1 = pltpu.async_copy(y_hbm.at[idx1], buf1, sem1)
        c0.wait()
        c1.wait()
        pltpu.sync_copy(buf0, yk_hbm.at[pl.ds(base, _CHUNK)])
        pltpu.sync_copy(buf1, yk_hbm.at[pl.ds(T + base, _CHUNK)])

    return dispatch, collect


def _mlp_body(be_ref, xg_ref, w1_ref, b1_ref, w2_ref, b2_ref, y_ref):
    xb = xg_ref[...].astype(jnp.bfloat16)
    h = jnp.dot(xb, w1_ref[0].astype(jnp.bfloat16),
                preferred_element_type=jnp.float32)
    h = h + b1_ref[0]
    h = h * 0.5 * (1.0 + jax.lax.erf(h * (2.0 ** -0.5)))
    y = jnp.dot(h.astype(jnp.bfloat16), w2_ref[0].astype(jnp.bfloat16),
                preferred_element_type=jnp.float32)
    y_ref[...] = y + b2_ref[0]


def _mlp_outer(be_ref, xg_hbm, w1_hbm, b1_hbm, w2_hbm, b2_hbm, y_hbm):
    wspec = functools.partial(pl.BlockSpec,
                              pipeline_mode=pl.Buffered(buffer_count=3,
                                                        use_lookahead=True))
    pltpu.emit_pipeline(
        _mlp_body_inner,
        grid=(G,),
        in_specs=[
            pl.BlockSpec((BM, D), lambda g: (g, 0)),
            wspec((1, D, H), lambda g: (be_ref[g], 0, 0)),
            wspec((1, 1, H), lambda g: (be_ref[g], 0, 0)),
            wspec((1, H, D), lambda g: (be_ref[g], 0, 0)),
            wspec((1, 1, D), lambda g: (be_ref[g], 0, 0)),
        ],
        out_specs=[pl.BlockSpec((BM, D), lambda g: (g, 0))],
    )(xg_hbm, w1_hbm, b1_hbm, w2_hbm, b2_hbm, y_hbm)


def _mlp_body_inner(xg_ref, w1_ref, b1_ref, w2_ref, b2_ref, y_ref):
    _mlp_body(None, xg_ref, w1_ref, b1_ref, w2_ref, b2_ref, y_ref)


def _mlp_call(be, xg, W1, b1r, W2, b2r):
    return pl.pallas_call(
        _mlp_outer,
        in_specs=[
            pl.BlockSpec(memory_space=pltpu.SMEM),
            pl.BlockSpec(memory_space=pltpu.HBM),
            pl.BlockSpec(memory_space=pltpu.HBM),
            pl.BlockSpec(memory_space=pltpu.HBM),
            pl.BlockSpec(memory_space=pltpu.HBM),
            pl.BlockSpec(memory_space=pltpu.HBM),
        ],
        out_specs=pl.BlockSpec(memory_space=pltpu.HBM),
        out_shape=jax.ShapeDtypeStruct((R, D), jnp.float32),
    )(be, xg, W1, b1r, W2, b2r)


def _combine_body(y0_ref, y1_ref, w_ref, o_ref):
    w = w_ref[...]
    o_ref[...] = (w[0].reshape(BM, 1) * y0_ref[...] +
                  w[1].reshape(BM, 1) * y1_ref[...])


def _combine_call(yk, wd):
    return pl.pallas_call(
        _combine_body,
        grid=(T // BM,),
        in_specs=[
            pl.BlockSpec((BM, D), lambda i: (i, 0)),
            pl.BlockSpec((BM, D), lambda i: (T // BM + i, 0)),
            pl.BlockSpec((K, BM), lambda i: (0, i)),
        ],
        out_specs=pl.BlockSpec((BM, D), lambda i: (i, 0)),
        out_shape=jax.ShapeDtypeStruct((T, D), jnp.float32),
    )(yk, yk, wd)


@jax.jit
def kernel(x, router_w, router_b, W1, b1, W2, b2):
    B, S, _ = x.shape
    xf = x.reshape(T, D)
    wT, destT, be128 = _router_call(xf, router_w, router_b.reshape(E, 1))
    be = be128[0, :G]
    dispatch, collect = _sc_kernels()
    xg = dispatch(xf, destT)
    y = _mlp_call(be, xg, W1, b1.reshape(E, 1, H), W2, b2.reshape(E, 1, D))
    yk = collect(y, destT)
    out = _combine_call(yk, wT)
    return out.reshape(B, S, D)


# speedup vs baseline: 1.0200x; 1.0200x over previous
import functools

import jax
import jax.numpy as jnp
from jax.experimental import pallas as pl
from jax.experimental.pallas import tpu as pltpu
from jax.experimental.pallas import tpu_sc as plsc

D, H, E, K = 768, 2048, 8, 2
T = 2048
BM = 256
G = T * K // BM + E
R = G * BM
CH = 256
SCW = 128


def _router_body(x_ref, rw_ref, rb_ref, w_ref, dest_ref, be_ref, p_ref):
    lg = jnp.dot(x_ref[...], rw_ref[...], preferred_element_type=jnp.float32)
    g = lg.T + rb_ref[...]
    m = jnp.max(g, axis=0, keepdims=True)
    ex = jnp.exp(g - m)
    g = ex / jnp.sum(ex, axis=0, keepdims=True)

    row = jax.lax.broadcasted_iota(jnp.int32, (E, T), 0)
    m1 = jnp.max(g, axis=0, keepdims=True)
    i1 = jnp.min(jnp.where(g == m1, row, E), axis=0, keepdims=True)
    oh1 = row == i1
    gm = jnp.where(oh1, -jnp.inf, g)
    m2 = jnp.max(gm, axis=0, keepdims=True)
    i2 = jnp.min(jnp.where(gm == m2, row, E), axis=0, keepdims=True)
    oh2 = row == i2
    maskf = oh1.astype(jnp.float32) + oh2.astype(jnp.float32)

    r_io = jax.lax.broadcasted_iota(jnp.int32, (CH, CH), 0)
    c_io = jax.lax.broadcasted_iota(jnp.int32, (CH, CH), 1)
    tri = (r_io < c_io).astype(jnp.float32)
    tot = jnp.zeros((E, 1), jnp.float32)
    for i in range(T // CH):
        mblk = maskf[:, i * CH:(i + 1) * CH]
        p_ref[:, i * CH:(i + 1) * CH] = (
            jnp.dot(mblk, tri, preferred_element_type=jnp.float32) + tot)
        tot = tot + jnp.sum(mblk, axis=1, keepdims=True)

    counts = tot
    nbf = jnp.floor((counts + float(BM - 1)) * (1.0 / BM))
    r8 = jax.lax.broadcasted_iota(jnp.int32, (E, E), 0)
    c8 = jax.lax.broadcasted_iota(jnp.int32, (E, E), 1)
    l8 = (c8 <= r8).astype(jnp.float32)
    cum = jnp.dot(l8, nbf, preferred_element_type=jnp.float32)
    bsr = (cum - nbf) * float(BM)

    destf = bsr + p_ref[...]
    dest_ref[0:1, :] = jnp.sum(
        jnp.where(oh1, destf, 0.0), axis=0, keepdims=True).astype(jnp.int32)
    dest_ref[1:2, :] = jnp.sum(
        jnp.where(oh2, destf, 0.0), axis=0, keepdims=True).astype(jnp.int32)
    w_ref[0:1, :] = m1
    w_ref[1:2, :] = m2

    g_io = jax.lax.broadcasted_iota(jnp.int32, (1, 128), 1)
    cum_i = cum.astype(jnp.int32)
    be_acc = jnp.zeros((1, 128), jnp.int32)
    for e in range(E):
        be_acc = be_acc + (g_io >= cum_i[e:e + 1, 0:1]).astype(jnp.int32)
    be_ref[...] = jnp.minimum(be_acc, E - 1)


def _router_call(xf, rw, rb):
    return pl.pallas_call(
        _router_body,
        out_shape=[
            jax.ShapeDtypeStruct((K, T), jnp.float32),
            jax.ShapeDtypeStruct((K, T), jnp.int32),
            jax.ShapeDtypeStruct((1, 128), jnp.int32),
        ],
        scratch_shapes=[pltpu.VMEM((E, T), jnp.float32)],
    )(xf, rw, rb)


_NW = 32
_CHUNK = T // _NW


@functools.lru_cache(maxsize=1)
def _sc_kernels():
    mesh = plsc.VectorSubcoreMesh(core_axis_name="c", subcore_axis_name="s")

    @functools.partial(
        pl.kernel,
        out_type=jax.ShapeDtypeStruct((R, D), jnp.float32),
        mesh=mesh,
        scratch_types=[
            pltpu.VMEM((_CHUNK, D), jnp.float32),
            pltpu.VMEM((_CHUNK,), jnp.int32),
            pltpu.VMEM((_CHUNK,), jnp.int32),
            pltpu.SemaphoreType.DMA,
            pltpu.SemaphoreType.DMA,
        ])
    def dispatch(x_hbm, destT_hbm, xg_hbm, buf, idx0, idx1, sem0, sem1):
        wid = jax.lax.axis_index("s") * 2 + jax.lax.axis_index("c")
        base = wid * _CHUNK
        pltpu.sync_copy(destT_hbm.at[0, pl.ds(base, _CHUNK)], idx0)
        pltpu.sync_copy(destT_hbm.at[1, pl.ds(base, _CHUNK)], idx1)
        pltpu.sync_copy(x_hbm.at[pl.ds(base, _CHUNK)], buf)
        c0 = pltpu.async_copy(buf, xg_hbm.at[idx0], sem0)
        c1 = pltpu.async_copy(buf, xg_hbm.at[idx1], sem1)
        c0.wait()
        c1.wait()

    @functools.partial(
        pl.kernel,
        out_type=jax.ShapeDtypeStruct((K * T, D), jnp.float32),
        mesh=mesh,
        scratch_types=[
            pltpu.VMEM((_CHUNK, D), jnp.float32),
            pltpu.VMEM((_CHUNK, D), jnp.float32),
            pltpu.VMEM((_CHUNK,), jnp.int32),
            pltpu.VMEM((_CHUNK,), jnp.int32),
            pltpu.SemaphoreType.DMA,
            pltpu.SemaphoreType.DMA,
        ])
    def collect(y_hbm, destT_hbm, yk_hbm, buf0, buf1, idx0, idx1, sem0, sem1):
        wid = jax.lax.axis_index("s") * 2 + jax.lax.axis_index("c")
        base = wid * _CHUNK
        pltpu.sync_copy(destT_hbm.at[0, pl.ds(base, _CHUNK)], idx0)
        pltpu.sync_copy(destT_hbm.at[1, pl.ds(base, _CHUNK)], idx1)
        c0 = pltpu.async_copy(y_hbm.at[idx0], buf0, sem0)
        c1 = pltpu.async_copy(y_hbm.at[idx1], buf1, sem1)
        c0.wait()
        c1.wait()
        pltpu.sync_copy(buf0, yk_hbm.at[pl.ds(base, _CHUNK)])
        pltpu.sync_copy(buf1, yk_hbm.at[pl.ds(T + base, _CHUNK)])

    return dispatch, collect


def _mlp_body(be_ref, xg_ref, w1_ref, b1_ref, w2_ref, b2_ref, y_ref):
    xb = xg_ref[...].astype(jnp.bfloat16)
    h = jnp.dot(xb, w1_ref[0],
                preferred_element_type=jnp.float32)
    h = h + b1_ref[0]
    h = h * 0.5 * (1.0 + jax.lax.erf(h * (2.0 ** -0.5)))
    y = jnp.dot(h.astype(jnp.bfloat16), w2_ref[0].astype(jnp.bfloat16),
                preferred_element_type=jnp.float32)
    y_ref[...] = y + b2_ref[0]


def _mlp_outer(be_ref, xg_hbm, w1_hbm, b1_hbm, w2_hbm, b2_hbm, y_hbm):
    wspec = functools.partial(pl.BlockSpec,
                              pipeline_mode=pl.Buffered(buffer_count=3,
                                                        use_lookahead=True))
    pltpu.emit_pipeline(
        _mlp_body_inner,
        grid=(G,),
        in_specs=[
            pl.BlockSpec((BM, D), lambda g: (g, 0)),
            wspec((1, D, H), lambda g: (be_ref[g], 0, 0)),
            wspec((1, 1, H), lambda g: (be_ref[g], 0, 0)),
            wspec((1, H, D), lambda g: (be_ref[g], 0, 0)),
            wspec((1, 1, D), lambda g: (be_ref[g], 0, 0)),
        ],
        out_specs=[pl.BlockSpec((BM, D), lambda g: (g, 0))],
    )(xg_hbm, w1_hbm, b1_hbm, w2_hbm, b2_hbm, y_hbm)


def _mlp_body_inner(xg_ref, w1_ref, b1_ref, w2_ref, b2_ref, y_ref):
    _mlp_body(None, xg_ref, w1_ref, b1_ref, w2_ref, b2_ref, y_ref)


def _mlp_call(be, xg, W1, b1r, W2, b2r):
    return pl.pallas_call(
        _mlp_outer,
        in_specs=[
            pl.BlockSpec(memory_space=pltpu.SMEM),
            pl.BlockSpec(memory_space=pltpu.HBM),
            pl.BlockSpec(memory_space=pltpu.HBM),
            pl.BlockSpec(memory_space=pltpu.HBM),
            pl.BlockSpec(memory_space=pltpu.HBM),
            pl.BlockSpec(memory_space=pltpu.HBM),
        ],
        out_specs=pl.BlockSpec(memory_space=pltpu.HBM),
        out_shape=jax.ShapeDtypeStruct((R, D), jnp.float32),
    )(be, xg, W1, b1r, W2, b2r)


def _combine_body(y0_ref, y1_ref, w_ref, o_ref):
    w = w_ref[...]
    o_ref[...] = (w[0].reshape(BM, 1) * y0_ref[...] +
                  w[1].reshape(BM, 1) * y1_ref[...])


def _combine_call(yk, wd):
    return pl.pallas_call(
        _combine_body,
        grid=(T // BM,),
        in_specs=[
            pl.BlockSpec((BM, D), lambda i: (i, 0)),
            pl.BlockSpec((BM, D), lambda i: (T // BM + i, 0)),
            pl.BlockSpec((K, BM), lambda i: (0, i)),
        ],
        out_specs=pl.BlockSpec((BM, D), lambda i: (i, 0)),
        out_shape=jax.ShapeDtypeStruct((T, D), jnp.float32),
    )(yk, yk, wd)


@jax.jit
def kernel(x, router_w, router_b, W1, b1, W2, b2):
    B, S, _ = x.shape
    xf = x.reshape(T, D)
    wT, destT, be128 = _router_call(xf, router_w, router_b.reshape(E, 1))
    be = be128[0, :G]
    dispatch, collect = _sc_kernels()
    xg = dispatch(xf, destT)
    W1d, _ = jax.lax.optimization_barrier((W1, destT))
    W1b = W1d.astype(jnp.bfloat16)
    y = _mlp_call(be, xg, W1b, b1.reshape(E, 1, H), W2, b2.reshape(E, 1, D))
    yk = collect(y, destT)
    out = _combine_call(yk, wT)
    return out.reshape(B, S, D)


# speedup vs baseline: 1.1497x; 1.1272x over previous
import functools

import jax
import jax.numpy as jnp
from jax.experimental import pallas as pl
from jax.experimental.pallas import tpu as pltpu
from jax.experimental.pallas import tpu_sc as plsc

D, H, E, K = 768, 2048, 8, 2
T = 2048
BM = 256
G = T * K // BM + E
R = G * BM
CH = 256
SCW = 128


def _router_body(x_ref, rw_ref, rb_ref, w_ref, dest_ref, be_ref, p_ref):
    lg = jnp.dot(x_ref[...], rw_ref[...], preferred_element_type=jnp.float32)
    g = lg.T + rb_ref[...]
    m = jnp.max(g, axis=0, keepdims=True)
    ex = jnp.exp(g - m)
    g = ex / jnp.sum(ex, axis=0, keepdims=True)

    row = jax.lax.broadcasted_iota(jnp.int32, (E, T), 0)
    m1 = jnp.max(g, axis=0, keepdims=True)
    i1 = jnp.min(jnp.where(g == m1, row, E), axis=0, keepdims=True)
    oh1 = row == i1
    gm = jnp.where(oh1, -jnp.inf, g)
    m2 = jnp.max(gm, axis=0, keepdims=True)
    i2 = jnp.min(jnp.where(gm == m2, row, E), axis=0, keepdims=True)
    oh2 = row == i2
    maskf = oh1.astype(jnp.float32) + oh2.astype(jnp.float32)

    r_io = jax.lax.broadcasted_iota(jnp.int32, (CH, CH), 0)
    c_io = jax.lax.broadcasted_iota(jnp.int32, (CH, CH), 1)
    tri = (r_io < c_io).astype(jnp.float32)
    tot = jnp.zeros((E, 1), jnp.float32)
    for i in range(T // CH):
        mblk = maskf[:, i * CH:(i + 1) * CH]
        p_ref[:, i * CH:(i + 1) * CH] = (
            jnp.dot(mblk, tri, preferred_element_type=jnp.float32) + tot)
        tot = tot + jnp.sum(mblk, axis=1, keepdims=True)

    counts = tot
    nbf = jnp.floor((counts + float(BM - 1)) * (1.0 / BM))
    r8 = jax.lax.broadcasted_iota(jnp.int32, (E, E), 0)
    c8 = jax.lax.broadcasted_iota(jnp.int32, (E, E), 1)
    l8 = (c8 <= r8).astype(jnp.float32)
    cum = jnp.dot(l8, nbf, preferred_element_type=jnp.float32)
    bsr = (cum - nbf) * float(BM)

    destf = bsr + p_ref[...]
    dest_ref[0:1, :] = jnp.sum(
        jnp.where(oh1, destf, 0.0), axis=0, keepdims=True).astype(jnp.int32)
    dest_ref[1:2, :] = jnp.sum(
        jnp.where(oh2, destf, 0.0), axis=0, keepdims=True).astype(jnp.int32)
    w_ref[0:1, :] = m1
    w_ref[1:2, :] = m2

    g_io = jax.lax.broadcasted_iota(jnp.int32, (1, 128), 1)
    cum_i = cum.astype(jnp.int32)
    be_acc = jnp.zeros((1, 128), jnp.int32)
    for e in range(E):
        be_acc = be_acc + (g_io >= cum_i[e:e + 1, 0:1]).astype(jnp.int32)
    be_ref[...] = jnp.minimum(be_acc, E - 1)


def _router_call(xf, rw, rb):
    return pl.pallas_call(
        _router_body,
        out_shape=[
            jax.ShapeDtypeStruct((K, T), jnp.float32),
            jax.ShapeDtypeStruct((K, T), jnp.int32),
            jax.ShapeDtypeStruct((1, 128), jnp.int32),
        ],
        scratch_shapes=[pltpu.VMEM((E, T), jnp.float32)],
    )(xf, rw, rb)


_NW = 32
_CHUNK = T // _NW


@functools.lru_cache(maxsize=1)
def _sc_kernels():
    mesh = plsc.VectorSubcoreMesh(core_axis_name="c", subcore_axis_name="s")

    @functools.partial(
        pl.kernel,
        out_type=jax.ShapeDtypeStruct((R, D), jnp.float32),
        mesh=mesh,
        scratch_types=[
            pltpu.VMEM((_CHUNK, D), jnp.float32),
            pltpu.VMEM((_CHUNK,), jnp.int32),
            pltpu.VMEM((_CHUNK,), jnp.int32),
            pltpu.SemaphoreType.DMA,
            pltpu.SemaphoreType.DMA,
        ])
    def dispatch(x_hbm, destT_hbm, xg_hbm, buf, idx0, idx1, sem0, sem1):
        wid = jax.lax.axis_index("s") * 2 + jax.lax.axis_index("c")
        base = wid * _CHUNK
        pltpu.sync_copy(destT_hbm.at[0, pl.ds(base, _CHUNK)], idx0)
        pltpu.sync_copy(destT_hbm.at[1, pl.ds(base, _CHUNK)], idx1)
        pltpu.sync_copy(x_hbm.at[pl.ds(base, _CHUNK)], buf)
        c0 = pltpu.async_copy(buf, xg_hbm.at[idx0], sem0)
        c1 = pltpu.async_copy(buf, xg_hbm.at[idx1], sem1)
        c0.wait()
        c1.wait()

    @functools.partial(
        pl.kernel,
        out_type=jax.ShapeDtypeStruct((K * T, D), jnp.float32),
        mesh=mesh,
        scratch_types=[
            pltpu.VMEM((_CHUNK, D), jnp.float32),
            pltpu.VMEM((_CHUNK, D), jnp.float32),
            pltpu.VMEM((_CHUNK,), jnp.int32),
            pltpu.VMEM((_CHUNK,), jnp.int32),
            pltpu.SemaphoreType.DMA,
            pltpu.SemaphoreType.DMA,
        ])
    def collect(y_hbm, destT_hbm, yk_hbm, buf0, buf1, idx0, idx1, sem0, sem1):
        wid = jax.lax.axis_index("s") * 2 + jax.lax.axis_index("c")
        base = wid * _CHUNK
        pltpu.sync_copy(destT_hbm.at[0, pl.ds(base, _CHUNK)], idx0)
        pltpu.sync_copy(destT_hbm.at[1, pl.ds(base, _CHUNK)], idx1)
        c0 = pltpu.async_copy(y_hbm.at[idx0], buf0, sem0)
        c1 = pltpu.async_copy(y_hbm.at[idx1], buf1, sem1)
        c0.wait()
        c1.wait()
        pltpu.sync_copy(buf0, yk_hbm.at[pl.ds(base, _CHUNK)])
        pltpu.sync_copy(buf1, yk_hbm.at[pl.ds(T + base, _CHUNK)])

    return dispatch, collect


def _mlp_outer(be_ref, xg_hbm, w1_hbm, b1_hbm, w2_hbm, b2_hbm, y_hbm,
               w1b, w2b, step_ref):
    step_ref[0] = 0
    step_ref[1] = -1

    def body(xg_ref, w1_ref, b1_ref, w2_ref, b2_ref, y_ref):
        i = step_ref[0]
        cur = be_ref[i]

        @pl.when(cur != step_ref[1])
        def _():
            w1b[...] = w1_ref[0].astype(jnp.bfloat16)
            w2b[...] = w2_ref[0].astype(jnp.bfloat16)

        step_ref[0] = i + 1
        step_ref[1] = cur

        xb = xg_ref[...].astype(jnp.bfloat16)
        h = jnp.dot(xb, w1b[...], preferred_element_type=jnp.float32)
        h = h + b1_ref[0]
        h = h * 0.5 * (1.0 + jax.lax.erf(h * (2.0 ** -0.5)))
        y = jnp.dot(h.astype(jnp.bfloat16), w2b[...],
                    preferred_element_type=jnp.float32)
        y_ref[...] = y + b2_ref[0]

    wspec = functools.partial(pl.BlockSpec,
                              pipeline_mode=pl.Buffered(buffer_count=3,
                                                        use_lookahead=True))
    pltpu.emit_pipeline(
        body,
        grid=(G,),
        in_specs=[
            pl.BlockSpec((BM, D), lambda g: (g, 0)),
            wspec((1, D, H), lambda g: (be_ref[g], 0, 0)),
            wspec((1, 1, H), lambda g: (be_ref[g], 0, 0)),
            wspec((1, H, D), lambda g: (be_ref[g], 0, 0)),
            wspec((1, 1, D), lambda g: (be_ref[g], 0, 0)),
        ],
        out_specs=[pl.BlockSpec((BM, D), lambda g: (g, 0))],
    )(xg_hbm, w1_hbm, b1_hbm, w2_hbm, b2_hbm, y_hbm)


def _mlp_call(be, xg, W1, b1r, W2, b2r):
    return pl.pallas_call(
        _mlp_outer,
        in_specs=[
            pl.BlockSpec(memory_space=pltpu.SMEM),
            pl.BlockSpec(memory_space=pltpu.HBM),
            pl.BlockSpec(memory_space=pltpu.HBM),
            pl.BlockSpec(memory_space=pltpu.HBM),
            pl.BlockSpec(memory_space=pltpu.HBM),
            pl.BlockSpec(memory_space=pltpu.HBM),
        ],
        out_specs=pl.BlockSpec(memory_space=pltpu.HBM),
        out_shape=jax.ShapeDtypeStruct((R, D), jnp.float32),
        scratch_shapes=[
            pltpu.VMEM((D, H), jnp.bfloat16),
            pltpu.VMEM((H, D), jnp.bfloat16),
            pltpu.SMEM((2,), jnp.int32),
        ],
    )(be, xg, W1, b1r, W2, b2r)


def _combine_body(y0_ref, y1_ref, w_ref, o_ref):
    w = w_ref[...]
    o_ref[...] = (w[0].reshape(BM, 1) * y0_ref[...] +
                  w[1].reshape(BM, 1) * y1_ref[...])


def _combine_call(yk, wd):
    return pl.pallas_call(
        _combine_body,
        grid=(T // BM,),
        in_specs=[
            pl.BlockSpec((BM, D), lambda i: (i, 0)),
            pl.BlockSpec((BM, D), lambda i: (T // BM + i, 0)),
            pl.BlockSpec((K, BM), lambda i: (0, i)),
        ],
        out_specs=pl.BlockSpec((BM, D), lambda i: (i, 0)),
        out_shape=jax.ShapeDtypeStruct((T, D), jnp.float32),
    )(yk, yk, wd)


@jax.jit
def kernel(x, router_w, router_b, W1, b1, W2, b2):
    B, S, _ = x.shape
    xf = x.reshape(T, D)
    wT, destT, be128 = _router_call(xf, router_w, router_b.reshape(E, 1))
    be = be128[0, :G]
    dispatch, collect = _sc_kernels()
    xg = dispatch(xf, destT)
    y = _mlp_call(be, xg, W1, b1.reshape(E, 1, H), W2, b2.reshape(E, 1, D))
    yk = collect(y, destT)
    out = _combine_call(yk, wT)
    return out.reshape(B, S, D)


# speedup vs baseline: 1.1658x; 1.0140x over previous
import functools

import jax
import jax.numpy as jnp
from jax.experimental import pallas as pl
from jax.experimental.pallas import tpu as pltpu
from jax.experimental.pallas import tpu_sc as plsc

D, H, E, K = 768, 2048, 8, 2
T = 2048
BM = 256
G = T * K // BM + E
R = G * BM
CH = 256
SCW = 128


def _router_body(x_ref, rw_ref, rb_ref, w_ref, dest_ref, be_ref, p_ref):
    lg = jnp.dot(x_ref[...], rw_ref[...], preferred_element_type=jnp.float32)
    g = lg.T + rb_ref[...]
    m = jnp.max(g, axis=0, keepdims=True)
    ex = jnp.exp(g - m)
    g = ex / jnp.sum(ex, axis=0, keepdims=True)

    row = jax.lax.broadcasted_iota(jnp.int32, (E, T), 0)
    m1 = jnp.max(g, axis=0, keepdims=True)
    i1 = jnp.min(jnp.where(g == m1, row, E), axis=0, keepdims=True)
    oh1 = row == i1
    gm = jnp.where(oh1, -jnp.inf, g)
    m2 = jnp.max(gm, axis=0, keepdims=True)
    i2 = jnp.min(jnp.where(gm == m2, row, E), axis=0, keepdims=True)
    oh2 = row == i2
    maskf = oh1.astype(jnp.float32) + oh2.astype(jnp.float32)

    r_io = jax.lax.broadcasted_iota(jnp.int32, (CH, CH), 0)
    c_io = jax.lax.broadcasted_iota(jnp.int32, (CH, CH), 1)
    tri = (r_io < c_io).astype(jnp.float32)
    tot = jnp.zeros((E, 1), jnp.float32)
    for i in range(T // CH):
        mblk = maskf[:, i * CH:(i + 1) * CH]
        p_ref[:, i * CH:(i + 1) * CH] = (
            jnp.dot(mblk, tri, preferred_element_type=jnp.float32) + tot)
        tot = tot + jnp.sum(mblk, axis=1, keepdims=True)

    counts = tot
    nbf = jnp.floor((counts + float(BM - 1)) * (1.0 / BM))
    r8 = jax.lax.broadcasted_iota(jnp.int32, (E, E), 0)
    c8 = jax.lax.broadcasted_iota(jnp.int32, (E, E), 1)
    l8 = (c8 <= r8).astype(jnp.float32)
    cum = jnp.dot(l8, nbf, preferred_element_type=jnp.float32)
    bsr = (cum - nbf) * float(BM)

    destf = bsr + p_ref[...]
    dest_ref[0:1, :] = jnp.sum(
        jnp.where(oh1, destf, 0.0), axis=0, keepdims=True).astype(jnp.int32)
    dest_ref[1:2, :] = jnp.sum(
        jnp.where(oh2, destf, 0.0), axis=0, keepdims=True).astype(jnp.int32)
    w_ref[0:1, :] = m1
    w_ref[1:2, :] = m2

    g_io = jax.lax.broadcasted_iota(jnp.int32, (1, 128), 1)
    cum_i = cum.astype(jnp.int32)
    be_acc = jnp.zeros((1, 128), jnp.int32)
    for e in range(E):
        be_acc = be_acc + (g_io >= cum_i[e:e + 1, 0:1]).astype(jnp.int32)
    be_ref[...] = jnp.minimum(be_acc, E - 1)


def _router_call(xf, rw, rb):
    return pl.pallas_call(
        _router_body,
        out_shape=[
            jax.ShapeDtypeStruct((K, T), jnp.float32),
            jax.ShapeDtypeStruct((K, T), jnp.int32),
            jax.ShapeDtypeStruct((1, 128), jnp.int32),
        ],
        scratch_shapes=[pltpu.VMEM((E, T), jnp.float32)],
    )(xf, rw, rb)


_NW = 32
_CHUNK = T // _NW


@functools.lru_cache(maxsize=1)
def _sc_kernels():
    mesh = plsc.VectorSubcoreMesh(core_axis_name="c", subcore_axis_name="s")

    @functools.partial(
        pl.kernel,
        out_type=jax.ShapeDtypeStruct((R, D), jnp.float32),
        mesh=mesh,
        scratch_types=[
            pltpu.VMEM((_CHUNK, D), jnp.float32),
            pltpu.VMEM((_CHUNK,), jnp.int32),
            pltpu.VMEM((_CHUNK,), jnp.int32),
            pltpu.SemaphoreType.DMA,
            pltpu.SemaphoreType.DMA,
            pltpu.SemaphoreType.DMA,
        ])
    def dispatch(x_hbm, destT_hbm, xg_hbm, buf, idx0, idx1, sem0, sem1, sem2):
        wid = jax.lax.axis_index("s") * 2 + jax.lax.axis_index("c")
        base = wid * _CHUNK
        l0 = pltpu.async_copy(destT_hbm.at[0, pl.ds(base, _CHUNK)], idx0, sem0)
        l1 = pltpu.async_copy(destT_hbm.at[1, pl.ds(base, _CHUNK)], idx1, sem1)
        lx = pltpu.async_copy(x_hbm.at[pl.ds(base, _CHUNK)], buf, sem2)
        l0.wait()
        l1.wait()
        lx.wait()
        c0 = pltpu.async_copy(buf, xg_hbm.at[idx0], sem0)
        c1 = pltpu.async_copy(buf, xg_hbm.at[idx1], sem1)
        c0.wait()
        c1.wait()

    @functools.partial(
        pl.kernel,
        out_type=jax.ShapeDtypeStruct((K * T, D), jnp.float32),
        mesh=mesh,
        scratch_types=[
            pltpu.VMEM((_CHUNK, D), jnp.float32),
            pltpu.VMEM((_CHUNK, D), jnp.float32),
            pltpu.VMEM((_CHUNK,), jnp.int32),
            pltpu.VMEM((_CHUNK,), jnp.int32),
            pltpu.SemaphoreType.DMA,
            pltpu.SemaphoreType.DMA,
        ])
    def collect(y_hbm, destT_hbm, yk_hbm, buf0, buf1, idx0, idx1, sem0, sem1):
        wid = jax.lax.axis_index("s") * 2 + jax.lax.axis_index("c")
        base = wid * _CHUNK
        l0 = pltpu.async_copy(destT_hbm.at[0, pl.ds(base, _CHUNK)], idx0, sem0)
        l1 = pltpu.async_copy(destT_hbm.at[1, pl.ds(base, _CHUNK)], idx1, sem1)
        l0.wait()
        l1.wait()
        c0 = pltpu.async_copy(y_hbm.at[idx0], buf0, sem0)
        c1 = pltpu.async_copy(y_hbm.at[idx1], buf1, sem1)
        c0.wait()
        s0 = pltpu.async_copy(buf0, yk_hbm.at[pl.ds(base, _CHUNK)], sem0)
        c1.wait()
        s1 = pltpu.async_copy(buf1, yk_hbm.at[pl.ds(T + base, _CHUNK)], sem1)
        s0.wait()
        s1.wait()

    return dispatch, collect


def _mlp_body(be_ref, xg_ref, w1_ref, b1_ref, w2_ref, b2_ref, y_ref):
    xb = xg_ref[...].astype(jnp.bfloat16)
    h = jnp.dot(xb, w1_ref[0].astype(jnp.bfloat16),
                preferred_element_type=jnp.float32)
    h = h + b1_ref[0]
    h = h * 0.5 * (1.0 + jax.lax.erf(h * (2.0 ** -0.5)))
    y = jnp.dot(h.astype(jnp.bfloat16), w2_ref[0].astype(jnp.bfloat16),
                preferred_element_type=jnp.float32)
    y_ref[...] = y + b2_ref[0]


def _mlp_outer(be_ref, xg_hbm, w1_hbm, b1_hbm, w2_hbm, b2_hbm, y_hbm):
    wspec = functools.partial(pl.BlockSpec,
                              pipeline_mode=pl.Buffered(buffer_count=3,
                                                        use_lookahead=True))
    pltpu.emit_pipeline(
        _mlp_body_inner,
        grid=(G,),
        in_specs=[
            pl.BlockSpec((BM, D), lambda g: (g, 0)),
            wspec((1, D, H), lambda g: (be_ref[g], 0, 0)),
            wspec((1, 1, H), lambda g: (be_ref[g], 0, 0)),
            wspec((1, H, D), lambda g: (be_ref[g], 0, 0)),
            wspec((1, 1, D), lambda g: (be_ref[g], 0, 0)),
        ],
        out_specs=[pl.BlockSpec((BM, D), lambda g: (g, 0))],
    )(xg_hbm, w1_hbm, b1_hbm, w2_hbm, b2_hbm, y_hbm)


def _mlp_body_inner(xg_ref, w1_ref, b1_ref, w2_ref, b2_ref, y_ref):
    _mlp_body(None, xg_ref, w1_ref, b1_ref, w2_ref, b2_ref, y_ref)


def _mlp_call(be, xg, W1, b1r, W2, b2r):
    return pl.pallas_call(
        _mlp_outer,
        in_specs=[
            pl.BlockSpec(memory_space=pltpu.SMEM),
            pl.BlockSpec(memory_space=pltpu.HBM),
            pl.BlockSpec(memory_space=pltpu.HBM),
            pl.BlockSpec(memory_space=pltpu.HBM),
            pl.BlockSpec(memory_space=pltpu.HBM),
            pl.BlockSpec(memory_space=pltpu.HBM),
        ],
        out_specs=pl.BlockSpec(memory_space=pltpu.HBM),
        out_shape=jax.ShapeDtypeStruct((R, D), jnp.float32),
    )(be, xg, W1, b1r, W2, b2r)


def _combine_body(y0_ref, y1_ref, w_ref, o_ref):
    w = w_ref[...]
    o_ref[...] = (w[0].reshape(BM, 1) * y0_ref[...] +
                  w[1].reshape(BM, 1) * y1_ref[...])


def _combine_call(yk, wd):
    return pl.pallas_call(
        _combine_body,
        grid=(T // BM,),
        in_specs=[
            pl.BlockSpec((BM, D), lambda i: (i, 0)),
            pl.BlockSpec((BM, D), lambda i: (T // BM + i, 0)),
            pl.BlockSpec((K, BM), lambda i: (0, i)),
        ],
        out_specs=pl.BlockSpec((BM, D), lambda i: (i, 0)),
        out_shape=jax.ShapeDtypeStruct((T, D), jnp.float32),
    )(yk, yk, wd)


@jax.jit
def kernel(x, router_w, router_b, W1, b1, W2, b2):
    B, S, _ = x.shape
    xf = x.reshape(T, D)
    wT, destT, be128 = _router_call(xf, router_w, router_b.reshape(E, 1))
    be = be128[0, :G]
    dispatch, collect = _sc_kernels()
    xg = dispatch(xf, destT)
    y = _mlp_call(be, xg, W1, b1.reshape(E, 1, H), W2, b2.reshape(E, 1, D))
    yk = collect(y, destT)
    out = _combine_call(yk, wT)
    return out.reshape(B, S, D)
